# baseline (speedup 1.0000x reference)
import functools

import jax
import jax.numpy as jnp
from jax import lax
from jax.experimental import pallas as pl
from jax.experimental.pallas import tpu as pltpu
from jax.experimental.pallas import tpu_sc as plsc

N_NODES = 10000
N_EDGES = 320000
D = 128
N_PAD = 10240
NC = 2
NS = 16
NW = NC * NS
CH = 128
NCHUNKS = N_EDGES // CH
DH = D // 2
_RPT = N_PAD // NS

_DEG_Q, _DEG_R = divmod(NCHUNKS, NW)
_AGG_Q, _AGG_R = divmod(NCHUNKS, NS)

_mesh = plsc.VectorSubcoreMesh(core_axis_name="c", subcore_axis_name="s")
_sc_params = pltpu.CompilerParams(use_tc_tiling_on_sc=False,
                                  needs_layout_passes=False)


@functools.partial(
    pl.kernel,
    out_type=jax.ShapeDtypeStruct((NC, N_PAD), jnp.float32),
    mesh=_mesh,
    scratch_types=[
        pltpu.VMEM((2, CH), jnp.int32),
        pltpu.VMEM((CH,), jnp.float32),
        pltpu.VMEM((_RPT,), jnp.float32),
        pltpu.VMEM_SHARED((N_PAD,), jnp.float32),
        pltpu.SemaphoreType.DMA,
        pltpu.SemaphoreType.DMA,
    ],
    compiler_params=_sc_params,
)
def _sc_degs(ei_hbm, out_hbm, dst_b, ones_v, zero_v, degs_sh, sem0, sem1):
    c = lax.axis_index("c")
    s = lax.axis_index("s")
    wid = c * NS + s
    r0 = s * _RPT
    nch = _DEG_Q + jnp.where(wid < _DEG_R, 1, 0)
    base = _DEG_Q * wid + jnp.minimum(wid, _DEG_R)

    def fill(i, carry):
        zero_v[pl.ds(i * 16, 16)] = jnp.zeros((16,), jnp.float32)
        return carry

    lax.fori_loop(0, _RPT // 16, fill, 0)
    for i in range(CH // 16):
        ones_v[pl.ds(i * 16, 16)] = jnp.ones((16,), jnp.float32)
    pltpu.sync_copy(zero_v, degs_sh.at[pl.ds(r0, _RPT)])

    def load_idx(chunk, slot, sem):
        pltpu.async_copy(ei_hbm.at[1, pl.ds((base + chunk) * CH, CH)],
                         dst_b.at[slot], sem)

    def wait_idx(sem):
        pltpu.make_async_copy(ei_hbm.at[1, pl.ds(0, CH)], dst_b.at[0], sem).wait()

    load_idx(0, 0, sem0)
    load_idx(1, 1, sem1)
    plsc.subcore_barrier()

    def body(j, carry):
        i0 = 2 * j
        i1 = i0 + 1
        wait_idx(sem0)
        pltpu.sync_copy(ones_v, degs_sh.at[dst_b.at[0]], add=True)

        @pl.when(i0 + 2 < nch)
        def _():
            load_idx(i0 + 2, 0, sem0)

        wait_idx(sem1)
        pltpu.sync_copy(ones_v, degs_sh.at[dst_b.at[1]], add=True)

        @pl.when(i1 + 2 < nch)
        def _():
            load_idx(i1 + 2, 1, sem1)

        return carry

    lax.fori_loop(0, _DEG_Q // 2, body, 0)

    @pl.when(nch > _DEG_Q)
    def _():
        wait_idx(sem0)
        pltpu.sync_copy(ones_v, degs_sh.at[dst_b.at[0]], add=True)

    plsc.subcore_barrier()
    pltpu.sync_copy(degs_sh.at[pl.ds(r0, _RPT)],
                    out_hbm.at[c, pl.ds(r0, _RPT)])


@functools.partial(
    pl.kernel,
    out_type=jax.ShapeDtypeStruct((N_NODES, D), jnp.float32),
    mesh=_mesh,
    scratch_types=[
        pltpu.VMEM((4, CH), jnp.int32),
        pltpu.VMEM((4, CH), jnp.int32),
        pltpu.VMEM((CH, DH), jnp.float32),
        pltpu.VMEM((CH, DH), jnp.float32),
        pltpu.VMEM((CH, DH), jnp.float32),
        pltpu.VMEM((CH, DH), jnp.float32),
        pltpu.VMEM((_RPT,), jnp.float32),
        pltpu.VMEM((_RPT,), jnp.float32),
        pltpu.VMEM((16, DH), jnp.float32),
        pltpu.VMEM_SHARED((N_PAD, DH), jnp.float32),
        pltpu.VMEM_SHARED((N_PAD, DH), jnp.float32),
        pltpu.SemaphoreType.DMA,
        pltpu.SemaphoreType.DMA,
        pltpu.SemaphoreType.DMA,
        pltpu.SemaphoreType.DMA,
        pltpu.SemaphoreType.DMA,
        pltpu.SemaphoreType.DMA,
        pltpu.SemaphoreType.DMA,
        pltpu.SemaphoreType.DMA,
        pltpu.SemaphoreType.DMA,
        pltpu.SemaphoreType.DMA,
    ],
    compiler_params=_sc_params,
)
def _sc_agg(xw_hbm, ei_hbm, degs_hbm, b_hbm, out_hbm,
            src_b, dst_b, b0, b1, b2, b3, norm_v, d1_v, bias_t, h_sh, acc_sh,
            g0, g1, g2, g3, s0, s1, s2, s3, sem_i01, sem_i23):
    c = lax.axis_index("c")
    s = lax.axis_index("s")
    r0 = s * _RPT
    nch = _AGG_Q + jnp.where(s < _AGG_R, 1, 0)
    base = _AGG_Q * s + jnp.minimum(s, _AGG_R)
    iota = lax.iota(jnp.int32, 16)

    def fillz(i, carry):
        for k in range(DH // 16):
            b0[i, pl.ds(k * 16, 16)] = jnp.zeros((16,), jnp.float32)
        return carry

    lax.fori_loop(0, CH, fillz, 0)
    for k in range(_RPT // CH):
        pltpu.sync_copy(b0, acc_sh.at[pl.ds(r0 + k * CH, CH)])

    pltpu.sync_copy(degs_hbm.at[0, pl.ds(r0, _RPT)], norm_v)
    pltpu.sync_copy(degs_hbm.at[1, pl.ds(r0, _RPT)], d1_v)

    def norm_body(i, carry):
        x = norm_v[pl.ds(i * 16, 16)] + d1_v[pl.ds(i * 16, 16)]
        xi = plsc.bitcast(x, jnp.int32)
        yi = jnp.int32(0x5F3759DF) - lax.shift_right_logical(xi, 1)
        y = plsc.bitcast(yi, jnp.float32)
        for _ in range(3):
            y = y * (1.5 - 0.5 * x * y * y)
        norm_v[pl.ds(i * 16, 16)] = jnp.where(x > 0.0, y, 0.0)
        return carry

    lax.fori_loop(0, _RPT // 16, norm_body, 0)

    pltpu.sync_copy(b_hbm.at[pl.ds(c * DH, DH)], bias_t.at[0])
    for g in range(DH // 16):
        v = bias_t[0, pl.ds(g * 16, 16)]
        for r in range(1, 16):
            bias_t[r, pl.ds(g * 16, 16)] = v

    def stage_body(k, carry):
        row = r0 + k * CH
        pltpu.sync_copy(xw_hbm.at[pl.ds(row, CH), pl.ds(c * DH, DH)], b1)

        def grp(g, carry2):
            norms = norm_v[pl.ds(k * CH + g * 16, 16)]
            ridx = iota + g * 16
            for col in range(DH):
                cidx = jnp.full((16,), col, jnp.int32)
                val = plsc.load_gather(b1, [ridx, cidx])
                plsc.store_scatter(b1, [ridx, cidx], val * norms)
            return carry2

        lax.fori_loop(0, CH // 16, grp, 0)
        pltpu.sync_copy(b1, h_sh.at[pl.ds(row, CH)])
        return carry

    lax.fori_loop(0, _RPT // CH, stage_body, 0)

    def load_idx(chunk, slot, sem):
        off = (base + chunk) * CH
        pltpu.async_copy(ei_hbm.at[0, pl.ds(off, CH)], src_b.at[slot], sem)
        pltpu.async_copy(ei_hbm.at[1, pl.ds(off, CH)], dst_b.at[slot], sem)

    def wait_idx(sem, n):
        for _ in range(n):
            pltpu.make_async_copy(ei_hbm.at[0, pl.ds(0, CH)],
                                  src_b.at[0], sem).wait()

    def gather(slot, buf, sem):
        pltpu.async_copy(h_sh.at[src_b.at[slot]], buf, sem)

    def wait_gather(buf, sem):
        pltpu.make_async_copy(h_sh.at[src_b.at[0]], buf, sem).wait()

    def scatter(slot, buf, sem):
        pltpu.async_copy(buf, acc_sh.at[dst_b.at[slot]], sem, add=True)

    def wait_scatter(buf, sem):
        pltpu.make_async_copy(buf, acc_sh.at[dst_b.at[0]], sem).wait()

    load_idx(0, 0, sem_i01)
    load_idx(1, 1, sem_i01)
    load_idx(2, 2, sem_i23)
    load_idx(3, 3, sem_i23)
    plsc.subcore_barrier()

    NJ = _AGG_Q // 4

    def body(j, carry):
        c0 = 4 * j
        wait_idx(sem_i01, 4)
        gather(0, b0, g0)
        gather(1, b1, g1)
        wait_idx(sem_i23, 4)
        gather(2, b2, g2)
        gather(3, b3, g3)
        wait_gather(b0, g0)
        scatter(0, b0, s0)
        wait_gather(b1, g1)
        scatter(1, b1, s1)
        wait_gather(b2, g2)
        scatter(2, b2, s2)
        wait_gather(b3, g3)
        scatter(3, b3, s3)
        wait_scatter(b0, s0)
        wait_scatter(b1, s1)

        @pl.when(c0 + 4 < nch)
        def _():
            load_idx(c0 + 4, 0, sem_i01)

        @pl.when(c0 + 5 < nch)
        def _():
            load_idx(c0 + 5, 1, sem_i01)

        wait_scatter(b2, s2)
        wait_scatter(b3, s3)

        @pl.when(c0 + 6 < nch)
        def _():
            load_idx(c0 + 6, 2, sem_i23)

        @pl.when(c0 + 7 < nch)
        def _():
            load_idx(c0 + 7, 3, sem_i23)

        return carry

    lax.fori_loop(0, NJ, body, 0)

    @pl.when(nch > _AGG_Q)
    def _():
        wait_idx(sem_i01, 2)
        gather(0, b0, g0)
        wait_gather(b0, g0)
        scatter(0, b0, s0)
        wait_scatter(b0, s0)

    plsc.subcore_barrier()

    nwk = jnp.where(s == NS - 1, 5, _RPT // 80)

    def out_body(k, carry):
        row = r0 + k * 80
        pltpu.sync_copy(acc_sh.at[pl.ds(row, 80)], b1.at[pl.ds(0, 80)])

        def grp(g, carry2):
            norms = norm_v[pl.ds(k * 80 + g * 16, 16)]
            ridx = iota + g * 16
            for col in range(DH):
                cidx = jnp.full((16,), col, jnp.int32)
                val = plsc.load_gather(b1, [ridx, cidx])
                bias = plsc.load_gather(bias_t, [iota, cidx])
                plsc.store_scatter(b1, [ridx, cidx], val * norms + bias)
            return carry2

        lax.fori_loop(0, 80 // 16, grp, 0)
        pltpu.sync_copy(b1.at[pl.ds(0, 80)],
                        out_hbm.at[pl.ds(row, 80), pl.ds(c * DH, DH)])
        return carry

    lax.fori_loop(0, nwk, out_body, 0)


def _tc_matmul_body(feat_ref, w_ref, xw_ref):
    xw_ref[:N_NODES] = lax.dot_general(
        feat_ref[...], w_ref[...],
        dimension_numbers=(((1,), (1,)), ((), ())),
        preferred_element_type=jnp.float32)
    xw_ref[N_NODES:] = jnp.zeros((N_PAD - N_NODES, D), jnp.float32)


def kernel(features, edge_index, W, b):
    features = features.astype(jnp.float32)
    ei = edge_index.astype(jnp.int32)

    xw = pl.pallas_call(
        _tc_matmul_body,
        out_shape=jax.ShapeDtypeStruct((N_PAD, D), jnp.float32),
    )(features, W.astype(jnp.float32))

    degs_p = _sc_degs(ei)
    out = _sc_agg(xw, ei, degs_p, b.astype(jnp.float32))
    return out


# speedup vs baseline: 1.5027x; 1.5027x over previous
import functools

import jax
import jax.numpy as jnp
from jax import lax
from jax.experimental import pallas as pl
from jax.experimental.pallas import tpu as pltpu
from jax.experimental.pallas import tpu_sc as plsc

N_NODES = 10000
N_EDGES = 320000
D = 128
N_PAD = 10240
NC = 2
NS = 16
NW = NC * NS
CH = 128
NCHUNKS = N_EDGES // CH
DH = D // 2
_RPT = N_PAD // NS

_DEG_Q, _DEG_R = divmod(NCHUNKS, NW)
_AGG_Q, _AGG_R = divmod(NCHUNKS, NS)

_mesh = plsc.VectorSubcoreMesh(core_axis_name="c", subcore_axis_name="s")
_sc_params = pltpu.CompilerParams(use_tc_tiling_on_sc=False)


@functools.partial(
    pl.kernel,
    out_type=jax.ShapeDtypeStruct((NC, N_PAD), jnp.float32),
    mesh=_mesh,
    scratch_types=[
        pltpu.VMEM((2, CH), jnp.int32),
        pltpu.VMEM((CH,), jnp.float32),
        pltpu.VMEM((_RPT,), jnp.float32),
        pltpu.VMEM_SHARED((N_PAD,), jnp.float32),
        pltpu.SemaphoreType.DMA,
        pltpu.SemaphoreType.DMA,
    ],
    compiler_params=_sc_params,
)
def _sc_degs(ei_hbm, out_hbm, dst_b, ones_v, zero_v, degs_sh, sem0, sem1):
    c = lax.axis_index("c")
    s = lax.axis_index("s")
    wid = c * NS + s
    r0 = s * _RPT
    nch = _DEG_Q + jnp.where(wid < _DEG_R, 1, 0)
    base = _DEG_Q * wid + jnp.minimum(wid, _DEG_R)

    def fill(i, carry):
        zero_v[pl.ds(i * 16, 16)] = jnp.zeros((16,), jnp.float32)
        return carry

    lax.fori_loop(0, _RPT // 16, fill, 0)
    for i in range(CH // 16):
        ones_v[pl.ds(i * 16, 16)] = jnp.ones((16,), jnp.float32)
    pltpu.sync_copy(zero_v, degs_sh.at[pl.ds(r0, _RPT)])

    def load_idx(chunk, slot, sem):
        pltpu.async_copy(ei_hbm.at[1, pl.ds((base + chunk) * CH, CH)],
                         dst_b.at[slot], sem)

    def wait_idx(sem):
        pltpu.make_async_copy(ei_hbm.at[1, pl.ds(0, CH)], dst_b.at[0], sem).wait()

    load_idx(0, 0, sem0)
    load_idx(1, 1, sem1)
    plsc.subcore_barrier()

    def body(j, carry):
        i0 = 2 * j
        i1 = i0 + 1
        wait_idx(sem0)
        pltpu.sync_copy(ones_v, degs_sh.at[dst_b.at[0]], add=True)

        @pl.when(i0 + 2 < nch)
        def _():
            load_idx(i0 + 2, 0, sem0)

        wait_idx(sem1)
        pltpu.sync_copy(ones_v, degs_sh.at[dst_b.at[1]], add=True)

        @pl.when(i1 + 2 < nch)
        def _():
            load_idx(i1 + 2, 1, sem1)

        return carry

    lax.fori_loop(0, _DEG_Q // 2, body, 0)

    @pl.when(nch > _DEG_Q)
    def _():
        wait_idx(sem0)
        pltpu.sync_copy(ones_v, degs_sh.at[dst_b.at[0]], add=True)

    plsc.subcore_barrier()
    pltpu.sync_copy(degs_sh.at[pl.ds(r0, _RPT)],
                    out_hbm.at[c, pl.ds(r0, _RPT)])


@functools.partial(
    pl.kernel,
    out_type=jax.ShapeDtypeStruct((N_PAD, D), jnp.float32),
    mesh=_mesh,
    scratch_types=[
        pltpu.VMEM((4, CH), jnp.int32),
        pltpu.VMEM((4, CH), jnp.int32),
        pltpu.VMEM((CH, DH), jnp.float32),
        pltpu.VMEM((CH, DH), jnp.float32),
        pltpu.VMEM((CH, DH), jnp.float32),
        pltpu.VMEM((CH, DH), jnp.float32),
        pltpu.VMEM_SHARED((N_PAD, DH), jnp.float32),
        pltpu.VMEM_SHARED((N_PAD, DH), jnp.float32),
        pltpu.SemaphoreType.DMA,
        pltpu.SemaphoreType.DMA,
        pltpu.SemaphoreType.DMA,
        pltpu.SemaphoreType.DMA,
        pltpu.SemaphoreType.DMA,
        pltpu.SemaphoreType.DMA,
        pltpu.SemaphoreType.DMA,
        pltpu.SemaphoreType.DMA,
        pltpu.SemaphoreType.DMA,
        pltpu.SemaphoreType.DMA,
    ],
    compiler_params=_sc_params,
)
def _sc_agg(h_hbm, ei_hbm, out_hbm,
            src_b, dst_b, b0, b1, b2, b3, h_sh, acc_sh,
            g0, g1, g2, g3, s0, s1, s2, s3, sem_i01, sem_i23):
    c = lax.axis_index("c")
    s = lax.axis_index("s")
    r0 = s * _RPT
    bufs = (b0, b1, b2, b3)
    gsems = (g0, g1, g2, g3)
    ssems = (s0, s1, s2, s3)
    nch = _AGG_Q + jnp.where(s < _AGG_R, 1, 0)
    base = _AGG_Q * s + jnp.minimum(s, _AGG_R)

    def fill(i, carry):
        for k in range(DH // 16):
            b0[i, pl.ds(k * 16, 16)] = jnp.zeros((16,), jnp.float32)
        return carry

    lax.fori_loop(0, CH, fill, 0)
    for k in range(_RPT // CH):
        pltpu.sync_copy(b0, acc_sh.at[pl.ds(r0 + k * CH, CH)])
    pltpu.sync_copy(h_hbm.at[pl.ds(r0, _RPT), pl.ds(c * DH, DH)],
                    h_sh.at[pl.ds(r0, _RPT)])

    def load_idx(chunk, slot, sem):
        off = (base + chunk) * CH
        pltpu.async_copy(ei_hbm.at[0, pl.ds(off, CH)], src_b.at[slot], sem)
        pltpu.async_copy(ei_hbm.at[1, pl.ds(off, CH)], dst_b.at[slot], sem)

    def wait_idx(sem, n):
        for _ in range(n):
            pltpu.make_async_copy(ei_hbm.at[0, pl.ds(0, CH)],
                                  src_b.at[0], sem).wait()

    def gather(slot, buf, sem):
        pltpu.async_copy(h_sh.at[src_b.at[slot]], buf, sem)

    def wait_gather(buf, sem):
        pltpu.make_async_copy(h_sh.at[src_b.at[0]], buf, sem).wait()

    def scatter(slot, buf, sem):
        pltpu.async_copy(buf, acc_sh.at[dst_b.at[slot]], sem, add=True)

    def wait_scatter(buf, sem):
        pltpu.make_async_copy(buf, acc_sh.at[dst_b.at[0]], sem).wait()

    load_idx(0, 0, sem_i01)
    load_idx(1, 1, sem_i01)
    load_idx(2, 2, sem_i23)
    load_idx(3, 3, sem_i23)
    plsc.subcore_barrier()

    NJ = _AGG_Q // 4

    def body(j, carry):
        c0 = 4 * j
        wait_idx(sem_i01, 4)
        gather(0, b0, g0)
        gather(1, b1, g1)
        wait_idx(sem_i23, 4)
        gather(2, b2, g2)
        gather(3, b3, g3)
        wait_gather(b0, g0)
        scatter(0, b0, s0)
        wait_gather(b1, g1)
        scatter(1, b1, s1)
        wait_gather(b2, g2)
        scatter(2, b2, s2)
        wait_gather(b3, g3)
        scatter(3, b3, s3)
        wait_scatter(b0, s0)
        wait_scatter(b1, s1)

        @pl.when(c0 + 4 < nch)
        def _():
            load_idx(c0 + 4, 0, sem_i01)

        @pl.when(c0 + 5 < nch)
        def _():
            load_idx(c0 + 5, 1, sem_i01)

        wait_scatter(b2, s2)
        wait_scatter(b3, s3)

        @pl.when(c0 + 6 < nch)
        def _():
            load_idx(c0 + 6, 2, sem_i23)

        @pl.when(c0 + 7 < nch)
        def _():
            load_idx(c0 + 7, 3, sem_i23)

        return carry

    lax.fori_loop(0, NJ, body, 0)

    @pl.when(nch > _AGG_Q)
    def _():
        wait_idx(sem_i01, 2)
        gather(0, b0, g0)
        wait_gather(b0, g0)
        scatter(0, b0, s0)
        wait_scatter(b0, s0)

    plsc.subcore_barrier()
    pltpu.sync_copy(acc_sh.at[pl.ds(r0, _RPT)],
                    out_hbm.at[pl.ds(r0, _RPT), pl.ds(c * DH, DH)])


def _tc_prescale_body(degs_ref, feat_ref, h_ref):
    d = degs_ref[0] + degs_ref[1]
    norm = jnp.where(d > 0.0, lax.rsqrt(d), 0.0)
    h_ref[:N_NODES] = feat_ref[...] * norm[:N_NODES]
    h_ref[N_NODES:] = jnp.zeros((N_PAD - N_NODES, D), jnp.float32)


def _tc_finish_body(agg_ref, degs_ref, w_ref, b_ref, out_ref):
    a = agg_ref[:N_NODES]
    d = degs_ref[0, :N_NODES] + degs_ref[1, :N_NODES]
    norm = jnp.where(d > 0.0, lax.rsqrt(d), 0.0)
    h2 = a * norm
    out = lax.dot_general(h2, w_ref[...],
                          dimension_numbers=(((1,), (1,)), ((), ())),
                          preferred_element_type=jnp.float32)
    out_ref[...] = out + b_ref[...]


def kernel(features, edge_index, W, b):
    features = features.astype(jnp.float32)
    ei = edge_index.astype(jnp.int32)

    degs_p = _sc_degs(ei)
    degs_p3 = degs_p[:, :, None]

    h = pl.pallas_call(
        _tc_prescale_body,
        out_shape=jax.ShapeDtypeStruct((N_PAD, D), jnp.float32),
    )(degs_p3, features)

    agg = _sc_agg(h, ei)

    out = pl.pallas_call(
        _tc_finish_body,
        out_shape=jax.ShapeDtypeStruct((N_NODES, D), jnp.float32),
    )(agg, degs_p3, W.astype(jnp.float32), b.reshape(1, D).astype(jnp.float32))
    return out
